# baseline (speedup 1.0000x reference)
import functools

import jax
import jax.numpy as jnp
from jax import lax
from jax.experimental import pallas as pl
from jax.experimental.pallas import tpu as pltpu
from jax.experimental.pallas import tpu_sc as plsc

EMB = 32
NC, NS = 2, 16
NW = NC * NS
B = 16384
BPW = B // NW
CH = 128
NCH = BPW // CH
E = 1_000_000
HALF = EMB // 2


NTC = E // 128
_mesh_d = plsc.VectorSubcoreMesh(core_axis_name="c", subcore_axis_name="s")


@functools.partial(
    pl.kernel,
    out_type=[jax.ShapeDtypeStruct((E, EMB), jnp.float32)] * 3,
    mesh=_mesh_d,
    compiler_params=pltpu.CompilerParams(needs_layout_passes=False,
                                         use_tc_tiling_on_sc=True),
    scratch_types=[
        pltpu.VMEM((4, 8, 128), jnp.float32),
        pltpu.VMEM((128, EMB), jnp.float32),
    ],
)
def _detile(ent_t, relg_t, hyper_t, ent_o, relg_o, hyper_o, vbuf, obuf):
    wid = lax.axis_index("s") * NC + lax.axis_index("c")
    lanes = lax.broadcasted_iota(jnp.int32, (16,), 0)
    pairs = ((ent_t, ent_o), (relg_t, relg_o), (hyper_t, hyper_o))

    def do_col(col, n):
        off = pl.multiple_of(col * 128, 128)
        for src, dst in pairs:
            src3 = src.reshape(4, 8, E)
            pltpu.sync_copy(src3.at[:, :, pl.ds(off, n)],
                            vbuf.at[:, :, pl.ds(0, n)])
            for e0 in range(0, n, 16):
                esl = pl.ds(e0, 16)
                rows = e0 + lanes
                for d in range(EMB):
                    plsc.store_scatter(obuf, [rows, jnp.full((16,), d, jnp.int32)],
                                       vbuf[d // 8, d % 8, esl])
            pltpu.sync_copy(obuf.at[pl.ds(0, n)], dst.at[pl.ds(off, n)])

    ncols = (NTC - wid + NW - 1) // NW

    def col_body(k, carry):
        do_col(wid + k * NW, 128)
        return carry

    lax.fori_loop(0, ncols, col_body, 0)

    @pl.when(wid == 0)
    def _():
        do_col(NTC, 64)


def _relayout(ent_t, relg_t, hyper_t):
    return _detile(ent_t, relg_t, hyper_t)


_mesh = plsc.VectorSubcoreMesh(core_axis_name="c", subcore_axis_name="s")


@functools.partial(
    pl.kernel,
    out_type=jax.ShapeDtypeStruct((B, EMB), jnp.float32),
    mesh=_mesh,
    compiler_params=pltpu.CompilerParams(needs_layout_passes=False,
                                         use_tc_tiling_on_sc=False),
    scratch_types=[
        pltpu.VMEM((NCH, CH), jnp.int32),
        pltpu.VMEM((NCH, CH), jnp.int32),
        pltpu.VMEM((NCH, CH), jnp.int32),
        pltpu.VMEM((BPW, EMB), jnp.float32),
        pltpu.VMEM((BPW, EMB), jnp.float32),
        pltpu.VMEM((BPW, EMB), jnp.float32),
        pltpu.VMEM((BPW, EMB), jnp.float32),
        pltpu.VMEM((BPW, EMB), jnp.float32),
        pltpu.SemaphoreType.DMA,
    ],
)
def _transh_sc(head_hbm, rel_hbm, tail_hbm, ent_hbm, relg_hbm, hyper_hbm,
               out_hbm, hidx, ridx, tidx, hrows, trows, yrows, grows, orows,
               sem):
    wid = lax.axis_index("s") * NC + lax.axis_index("c")
    base = wid * BPW
    cbase = wid * NCH

    pltpu.sync_copy(head_hbm.at[pl.ds(cbase, NCH)], hidx)
    pltpu.sync_copy(rel_hbm.at[pl.ds(cbase, NCH)], ridx)
    pltpu.sync_copy(tail_hbm.at[pl.ds(cbase, NCH)], tidx)

    copies = []
    for c in range(NCH):
        dst = pl.ds(c * CH, CH)
        copies.append(pltpu.async_copy(ent_hbm.at[hidx.at[c]],
                                       hrows.at[dst], sem))
        copies.append(pltpu.async_copy(ent_hbm.at[tidx.at[c]],
                                       trows.at[dst], sem))
        copies.append(pltpu.async_copy(hyper_hbm.at[ridx.at[c]],
                                       yrows.at[dst], sem))
        copies.append(pltpu.async_copy(relg_hbm.at[ridx.at[c]],
                                       grows.at[dst], sem))
    for cp in copies:
        cp.wait()

    lanes = lax.broadcasted_iota(jnp.int32, (HALF,), 0)

    def grp_body(grp, carry):
        rows = grp * HALF + lanes
        acc = jnp.zeros((HALF,), jnp.float32)
        for j in range(EMB):
            jv = jnp.full((HALF,), j, jnp.int32)
            h = plsc.load_gather(hrows, [rows, jv])
            t = plsc.load_gather(trows, [rows, jv])
            y = plsc.load_gather(yrows, [rows, jv])
            acc = acc + (h - t) * y
        for j in range(EMB):
            jv = jnp.full((HALF,), j, jnp.int32)
            h = plsc.load_gather(hrows, [rows, jv])
            t = plsc.load_gather(trows, [rows, jv])
            y = plsc.load_gather(yrows, [rows, jv])
            g = plsc.load_gather(grows, [rows, jv])
            plsc.store_scatter(orows, [rows, jv], (h - t) + g - y * acc)
        return carry

    lax.fori_loop(0, BPW // HALF, grp_body, 0)

    pltpu.sync_copy(orows, out_hbm.at[pl.ds(base, BPW)])


def kernel(in_triple, ent_emb, rel_emb, rel_hyper):
    head = in_triple[:, 0].astype(jnp.int32).reshape(NW * NCH, CH)
    rel = in_triple[:, 1].astype(jnp.int32).reshape(NW * NCH, CH)
    tail = in_triple[:, 2].astype(jnp.int32).reshape(NW * NCH, CH)
    ent_r, relg_r, hyper_r = _relayout(ent_emb.T, rel_emb.T, rel_hyper.T)
    return _transh_sc(head, rel, tail, ent_r, relg_r, hyper_r)


# speedup vs baseline: 3.0472x; 3.0472x over previous
import functools

import jax
import jax.numpy as jnp
from jax import lax
from jax.experimental import pallas as pl
from jax.experimental.pallas import tpu as pltpu
from jax.experimental.pallas import tpu_sc as plsc

EMB = 32
NC, NS = 2, 16
NW = NC * NS
B = 16384
BPW = B // NW
CH = 128
NCH = BPW // CH
HALF = EMB // 2

_mesh = plsc.VectorSubcoreMesh(core_axis_name="c", subcore_axis_name="s")


@functools.partial(
    pl.kernel,
    out_type=jax.ShapeDtypeStruct((B, EMB), jnp.float32),
    mesh=_mesh,
    compiler_params=pltpu.CompilerParams(needs_layout_passes=False,
                                         use_tc_tiling_on_sc=False),
    scratch_types=[
        pltpu.VMEM((NCH, CH), jnp.int32),
        pltpu.VMEM((NCH, CH), jnp.int32),
        pltpu.VMEM((NCH, CH), jnp.int32),
        pltpu.VMEM((BPW, EMB), jnp.float32),
        pltpu.VMEM((BPW, EMB), jnp.float32),
        pltpu.VMEM((BPW, EMB), jnp.float32),
        pltpu.VMEM((BPW, EMB), jnp.float32),
        pltpu.VMEM((BPW, EMB), jnp.float32),
        pltpu.SemaphoreType.DMA,
    ],
)
def _transh_sc(head_hbm, rel_hbm, tail_hbm, ent_hbm, relg_hbm, hyper_hbm,
               out_hbm, hidx, ridx, tidx, hrows, trows, yrows, grows, orows,
               sem):
    wid = lax.axis_index("s") * NC + lax.axis_index("c")
    base = wid * BPW
    cbase = wid * NCH

    pltpu.sync_copy(head_hbm.at[pl.ds(cbase, NCH)], hidx)
    pltpu.sync_copy(rel_hbm.at[pl.ds(cbase, NCH)], ridx)
    pltpu.sync_copy(tail_hbm.at[pl.ds(cbase, NCH)], tidx)

    copies = []
    for c in range(NCH):
        dst = pl.ds(c * CH, CH)
        copies.append(pltpu.async_copy(ent_hbm.at[hidx.at[c]],
                                       hrows.at[dst], sem))
        copies.append(pltpu.async_copy(ent_hbm.at[tidx.at[c]],
                                       trows.at[dst], sem))
        copies.append(pltpu.async_copy(hyper_hbm.at[ridx.at[c]],
                                       yrows.at[dst], sem))
        copies.append(pltpu.async_copy(relg_hbm.at[ridx.at[c]],
                                       grows.at[dst], sem))
    for cp in copies:
        cp.wait()

    lanes = lax.broadcasted_iota(jnp.int32, (HALF,), 0)

    def grp_body(grp, carry):
        rows = grp * HALF + lanes
        acc = jnp.zeros((HALF,), jnp.float32)
        for j in range(EMB):
            jv = jnp.full((HALF,), j, jnp.int32)
            h = plsc.load_gather(hrows, [rows, jv])
            t = plsc.load_gather(trows, [rows, jv])
            y = plsc.load_gather(yrows, [rows, jv])
            acc = acc + (h - t) * y
        for j in range(EMB):
            jv = jnp.full((HALF,), j, jnp.int32)
            h = plsc.load_gather(hrows, [rows, jv])
            t = plsc.load_gather(trows, [rows, jv])
            y = plsc.load_gather(yrows, [rows, jv])
            g = plsc.load_gather(grows, [rows, jv])
            plsc.store_scatter(orows, [rows, jv], (h - t) + g - y * acc)
        return carry

    lax.fori_loop(0, BPW // HALF, grp_body, 0)

    pltpu.sync_copy(orows, out_hbm.at[pl.ds(base, BPW)])


def kernel(in_triple, ent_emb, rel_emb, rel_hyper):
    head = in_triple[:, 0].astype(jnp.int32).reshape(NW * NCH, CH)
    rel = in_triple[:, 1].astype(jnp.int32).reshape(NW * NCH, CH)
    tail = in_triple[:, 2].astype(jnp.int32).reshape(NW * NCH, CH)
    return _transh_sc(head, rel, tail, ent_emb, rel_emb, rel_hyper)
